# baseline (speedup 1.0000x reference)
import jax
import jax.numpy as jnp
from jax.experimental import pallas as pl
from jax.experimental.pallas import tpu as pltpu


def _scale_kernel(x_ref, m_ref, o_ref):
    o_ref[...] = x_ref[...] * m_ref[...]


def kernel(x, mask):
    B, N, W = x.shape
    BB = 56
    m3 = mask.reshape(1, N, 1)
    return pl.pallas_call(
        _scale_kernel,
        grid=(B, N // BB),
        in_specs=[
            pl.BlockSpec((1, BB, W), lambda i, j: (i, j, 0)),
            pl.BlockSpec((1, BB, 1), lambda i, j: (0, j, 0)),
        ],
        out_specs=pl.BlockSpec((1, BB, W), lambda i, j: (i, j, 0)),
        out_shape=jax.ShapeDtypeStruct((B, N, W), x.dtype),
        compiler_params=pltpu.CompilerParams(
            dimension_semantics=("parallel", "parallel"),
        ),
    )(x, m3)


# speedup vs baseline: 1.0157x; 1.0157x over previous
import jax
import jax.numpy as jnp
from jax.experimental import pallas as pl
from jax.experimental.pallas import tpu as pltpu


def _scale_kernel(x_ref, m_ref, o_ref):
    o_ref[...] = x_ref[...] * m_ref[...]


def kernel(x, mask):
    B, N, W = x.shape
    BB = 112
    m3 = mask.reshape(1, N, 1)
    return pl.pallas_call(
        _scale_kernel,
        grid=(B, N // BB),
        in_specs=[
            pl.BlockSpec((1, BB, W), lambda i, j: (i, j, 0)),
            pl.BlockSpec((1, BB, 1), lambda i, j: (0, j, 0)),
        ],
        out_specs=pl.BlockSpec((1, BB, W), lambda i, j: (i, j, 0)),
        out_shape=jax.ShapeDtypeStruct((B, N, W), x.dtype),
        compiler_params=pltpu.CompilerParams(
            dimension_semantics=("parallel", "parallel"),
        ),
    )(x, m3)


# speedup vs baseline: 1.0290x; 1.0131x over previous
import jax
import jax.numpy as jnp
from jax.experimental import pallas as pl
from jax.experimental.pallas import tpu as pltpu


def _scale_kernel(x_ref, m_ref, o_ref):
    o_ref[...] = x_ref[...] * m_ref[...]


def kernel(x, mask):
    B, N, W = x.shape
    BB = 224
    m3 = mask.reshape(1, N, 1)
    return pl.pallas_call(
        _scale_kernel,
        grid=(B, N // BB),
        in_specs=[
            pl.BlockSpec((1, BB, W), lambda i, j: (i, j, 0)),
            pl.BlockSpec((1, BB, 1), lambda i, j: (0, j, 0)),
        ],
        out_specs=pl.BlockSpec((1, BB, W), lambda i, j: (i, j, 0)),
        out_shape=jax.ShapeDtypeStruct((B, N, W), x.dtype),
        compiler_params=pltpu.CompilerParams(
            dimension_semantics=("parallel", "parallel"),
            vmem_limit_bytes=100 * 1024 * 1024,
        ),
    )(x, m3)


# speedup vs baseline: 1.0299x; 1.0009x over previous
import jax
import jax.numpy as jnp
from jax.experimental import pallas as pl
from jax.experimental.pallas import tpu as pltpu


def _scale_kernel(x_ref, m_ref, o_ref):
    o_ref[...] = x_ref[...] * m_ref[...]


def kernel(x, mask):
    B, N, W = x.shape
    m3 = mask.reshape(1, N, 1)
    return pl.pallas_call(
        _scale_kernel,
        grid=(B,),
        in_specs=[
            pl.BlockSpec((1, N, W), lambda i: (i, 0, 0)),
            pl.BlockSpec((1, N, 1), lambda i: (0, 0, 0)),
        ],
        out_specs=pl.BlockSpec((1, N, W), lambda i: (i, 0, 0)),
        out_shape=jax.ShapeDtypeStruct((B, N, W), x.dtype),
        compiler_params=pltpu.CompilerParams(
            dimension_semantics=("parallel",),
            vmem_limit_bytes=100 * 1024 * 1024,
        ),
    )(x, m3)


# speedup vs baseline: 1.0392x; 1.0090x over previous
import jax
import jax.numpy as jnp
from jax.experimental import pallas as pl
from jax.experimental.pallas import tpu as pltpu


def _scale_kernel(x_ref, m_ref, o_ref):
    o_ref[...] = x_ref[...] * m_ref[...][None, :, None]


def kernel(x, mask):
    B, N, W = x.shape
    return pl.pallas_call(
        _scale_kernel,
        grid=(B,),
        in_specs=[
            pl.BlockSpec((1, N, W), lambda i: (i, 0, 0)),
            pl.BlockSpec((N,), lambda i: (0,)),
        ],
        out_specs=pl.BlockSpec((1, N, W), lambda i: (i, 0, 0)),
        out_shape=jax.ShapeDtypeStruct((B, N, W), x.dtype),
        compiler_params=pltpu.CompilerParams(
            dimension_semantics=("parallel",),
            vmem_limit_bytes=100 * 1024 * 1024,
        ),
    )(x, mask)
